# baseline (speedup 1.0000x reference)
import jax
import jax.numpy as jnp
from jax import lax
from jax.experimental import pallas as pl
from jax.experimental.pallas import tpu as pltpu, tpu_sc as plsc

D = 1024
S = 8192
B = 4
NW = 32
ROWS_PER_W = S // NW
CHUNK = 16
N_CHUNKS = ROWS_PER_W // CHUNK
NJOBS = N_CHUNKS * B
NXB = 3
NPB = 3
CSL = D // 16


def _sc_body(x_hbm, pos_hbm, out_hbm, x_v, pos_v, sem_xl, sem_pl, sem_st):
    cid = lax.axis_index("c")
    sid = lax.axis_index("s")
    wid = sid * 2 + cid
    row_base = wid * ROWS_PER_W

    def rows(j):
        return j % B, row_base + (j // B) * CHUNK

    def start_xload(j):
        b, r0 = rows(j)
        pltpu.async_copy(x_hbm.at[b, pl.ds(r0, CHUNK), :], x_v.at[j % NXB],
                         sem_xl)

    def start_pload(t):
        pltpu.async_copy(pos_hbm.at[pl.ds(row_base + t * CHUNK, CHUNK), :],
                         pos_v.at[t % NPB], sem_pl)

    start_pload(0)
    start_pload(1)
    start_xload(0)
    start_xload(1)

    def job(j, _):
        t = j // B
        b = j % B
        cur = j % NXB

        pltpu.make_async_copy(x_hbm.at[0, pl.ds(0, CHUNK), :], x_v.at[cur],
                              sem_xl).wait()

        @pl.when(b == 0)
        def _():
            pltpu.make_async_copy(pos_hbm.at[pl.ds(0, CHUNK), :],
                                  pos_v.at[t % NPB], sem_pl).wait()

            @pl.when(t + 2 < N_CHUNKS)
            def _():
                start_pload(t + 2)

        @pl.when(j + NXB - 1 < NJOBS)
        def _():
            @pl.when(j >= 1)
            def _():
                pltpu.make_async_copy(x_v.at[(j + NXB - 1) % NXB],
                                      out_hbm.at[0, pl.ds(0, CHUNK), :],
                                      sem_st).wait()
            start_xload(j + NXB - 1)

        xv = x_v.at[cur]
        pv = pos_v.at[t % NPB]

        def add_loop(i, _):
            r = i // CSL
            c = (i % CSL) * 16
            sl = pl.ds(c, 16)
            xv[r, sl] = xv[r, sl] + pv[r, sl]
            return 0

        lax.fori_loop(0, CHUNK * CSL, add_loop, 0, unroll=8)

        _, r0 = rows(j)
        pltpu.async_copy(xv, out_hbm.at[b, pl.ds(r0, CHUNK), :], sem_st)
        return 0

    lax.fori_loop(0, NJOBS, job, 0)

    for _ in range(NXB):
        pltpu.make_async_copy(x_v.at[0], out_hbm.at[0, pl.ds(0, CHUNK), :],
                              sem_st).wait()


_sc_call = pl.kernel(
    _sc_body,
    out_type=jax.ShapeDtypeStruct((B, S, D), jnp.float32),
    mesh=plsc.VectorSubcoreMesh(core_axis_name="c", subcore_axis_name="s"),
    scratch_types=[
        pltpu.VMEM((NXB, CHUNK, D), jnp.float32),
        pltpu.VMEM((NPB, CHUNK, D), jnp.float32),
        pltpu.SemaphoreType.DMA,
        pltpu.SemaphoreType.DMA,
        pltpu.SemaphoreType.DMA,
    ],
    compiler_params=pltpu.CompilerParams(use_tc_tiling_on_sc=True),
)


def kernel(x, pos_table):
    return _sc_call(x, pos_table)


# speedup vs baseline: 2.4283x; 2.4283x over previous
import jax
import jax.numpy as jnp
from jax import lax
from jax.experimental import pallas as pl
from jax.experimental.pallas import tpu as pltpu, tpu_sc as plsc

D = 1024
S = 8192
B = 4
NW = 32
ROWS_PER_W = S // NW
CHUNK = 16
N_CHUNKS = ROWS_PER_W // CHUNK


def _sc_body(x_hbm, pos_hbm, out_hbm, x_v, pos_v, sem):
    cid = lax.axis_index("c")
    sid = lax.axis_index("s")
    wid = sid * 2 + cid
    row_base = wid * ROWS_PER_W

    def chunk_loop(t, _):
        r0 = row_base + t * CHUNK
        pltpu.sync_copy(pos_hbm.at[pl.ds(r0, CHUNK), :], pos_v)

        def batch_loop(b, _):
            pltpu.sync_copy(x_hbm.at[b, pl.ds(r0, CHUNK), :], x_v)

            pltpu.sync_copy(x_v, out_hbm.at[b, pl.ds(r0, CHUNK), :])
            return 0

        lax.fori_loop(0, B, batch_loop, 0)
        return 0

    lax.fori_loop(0, N_CHUNKS, chunk_loop, 0)


_sc_call = pl.kernel(
    _sc_body,
    out_type=jax.ShapeDtypeStruct((B, S, D), jnp.float32),
    mesh=plsc.VectorSubcoreMesh(core_axis_name="c", subcore_axis_name="s"),
    scratch_types=[
        pltpu.VMEM((CHUNK, D), jnp.float32),
        pltpu.VMEM((CHUNK, D), jnp.float32),
        pltpu.SemaphoreType.DMA,
    ],
    compiler_params=pltpu.CompilerParams(use_tc_tiling_on_sc=True),
)


def kernel(x, pos_table):
    return _sc_call(x, pos_table)


# speedup vs baseline: 3.2858x; 1.3531x over previous
import jax
import jax.numpy as jnp
from jax import lax
from jax.experimental import pallas as pl
from jax.experimental.pallas import tpu as pltpu, tpu_sc as plsc

D = 1024
S = 8192
B = 4
NW = 32
ROWS_PER_W = S // NW
CHUNK = 16
N_CHUNKS = ROWS_PER_W // CHUNK
NJOBS = N_CHUNKS * B
NXB = 4
CSL = D // 16


def _sc_body(x_hbm, pos_hbm, out_hbm, x_v, pos_v, sem_xl, sem_pl, sem_st):
    cid = lax.axis_index("c")
    sid = lax.axis_index("s")
    wid = sid * 2 + cid
    row_base = wid * ROWS_PER_W

    def xslice(j):
        return x_hbm.at[j % B, pl.ds(row_base + (j // B) * CHUNK, CHUNK), :]

    def oslice(j):
        return out_hbm.at[j % B, pl.ds(row_base + (j // B) * CHUNK, CHUNK), :]

    def start_xload(j, slot):
        pltpu.async_copy(xslice(j), x_v.at[slot], sem_xl)

    def start_pload(t, slot):
        pltpu.async_copy(pos_hbm.at[pl.ds(row_base + t * CHUNK, CHUNK), :],
                         pos_v.at[slot], sem_pl)

    def wait_xload(slot):
        pltpu.make_async_copy(x_hbm.at[0, pl.ds(0, CHUNK), :], x_v.at[slot],
                              sem_xl).wait()

    def wait_pload(slot):
        pltpu.make_async_copy(pos_hbm.at[pl.ds(0, CHUNK), :], pos_v.at[slot],
                              sem_pl).wait()

    def wait_store(slot):
        pltpu.make_async_copy(x_v.at[slot], out_hbm.at[0, pl.ds(0, CHUNK), :],
                              sem_st).wait()

    start_pload(0, 0)
    start_pload(1, 1)
    start_xload(0, 0)
    start_xload(1, 1)

    def outer(g, _):
        j0 = g * 8
        for u in range(8):
            j = j0 + u
            s = u % 4
            ps = u // 4

            if u == 0 or u == 4:
                wait_pload(ps)

            wait_xload(s)

            xv = x_v.at[s]
            pv = pos_v.at[ps]

            def add_loop(i, _):
                r = i // CSL
                c = (i % CSL) * 16
                sl = pl.ds(c, 16)
                xv[r, sl] = xv[r, sl] + pv[r, sl]
                return 0

            lax.fori_loop(0, CHUNK * CSL, add_loop, 0, unroll=8)

            pltpu.async_copy(xv, oslice(j), sem_st)

            if u == 3 or u == 7:
                t = g * 2 + u // 4
                @pl.when(t + 2 < N_CHUNKS)
                def _():
                    start_pload(t + 2, ps)

            @pl.when(j + 2 < NJOBS)
            def _():
                @pl.when(j >= 2)
                def _():
                    wait_store((u + 2) % 4)
                start_xload(j + 2, (u + 2) % 4)
        return 0

    lax.fori_loop(0, NJOBS // 8, outer, 0)

    for k in range(NXB):
        wait_store(k)


_sc_call = pl.kernel(
    _sc_body,
    out_type=jax.ShapeDtypeStruct((B, S, D), jnp.float32),
    mesh=plsc.VectorSubcoreMesh(core_axis_name="c", subcore_axis_name="s"),
    scratch_types=[
        pltpu.VMEM((NXB, CHUNK, D), jnp.float32),
        pltpu.VMEM((2, CHUNK, D), jnp.float32),
        pltpu.SemaphoreType.DMA,
        pltpu.SemaphoreType.DMA,
        pltpu.SemaphoreType.DMA,
    ],
    compiler_params=pltpu.CompilerParams(use_tc_tiling_on_sc=True),
)


def kernel(x, pos_table):
    return _sc_call(x, pos_table)


# speedup vs baseline: 3.2910x; 1.0016x over previous
import jax
import jax.numpy as jnp
from jax import lax
from jax.experimental import pallas as pl
from jax.experimental.pallas import tpu as pltpu, tpu_sc as plsc

D = 1024
S = 8192
B = 4
NW = 32
ROWS_PER_W = S // NW
CHUNK = 16
N_CHUNKS = ROWS_PER_W // CHUNK
NJOBS = N_CHUNKS * B
NXB = 5
LEAD = 3
CSL = D // 16


def _sc_body(x_hbm, pos_hbm, out_hbm, x_v, pos_v, sem_xl, sem_pl, sem_st):
    cid = lax.axis_index("c")
    sid = lax.axis_index("s")
    wid = sid * 2 + cid
    row_base = wid * ROWS_PER_W

    def xslice(j):
        return x_hbm.at[j % B, pl.ds(row_base + (j // B) * CHUNK, CHUNK), :]

    def oslice(j):
        return out_hbm.at[j % B, pl.ds(row_base + (j // B) * CHUNK, CHUNK), :]

    def start_xload(j):
        pltpu.async_copy(xslice(j), x_v.at[j % NXB], sem_xl)

    def start_pload(t):
        pltpu.async_copy(pos_hbm.at[pl.ds(row_base + t * CHUNK, CHUNK), :],
                         pos_v.at[t % 2], sem_pl)

    def wait_xload(slot):
        pltpu.make_async_copy(x_hbm.at[0, pl.ds(0, CHUNK), :], x_v.at[slot],
                              sem_xl).wait()

    def wait_pload(slot):
        pltpu.make_async_copy(pos_hbm.at[pl.ds(0, CHUNK), :], pos_v.at[slot],
                              sem_pl).wait()

    def wait_store(slot):
        pltpu.make_async_copy(x_v.at[slot], out_hbm.at[0, pl.ds(0, CHUNK), :],
                              sem_st).wait()

    start_pload(0)
    start_pload(1)
    for j in range(LEAD):
        start_xload(j)

    for j in range(NJOBS):
        t = j // B
        b = j % B
        s = j % NXB
        ps = t % 2

        if b == 0:
            wait_pload(ps)
        wait_xload(s)

        xv = x_v.at[s]
        pv = pos_v.at[ps]

        def add_loop(i, _):
            r = i // CSL
            c = (i % CSL) * 16
            sl = pl.ds(c, 16)
            xv[r, sl] = xv[r, sl] + pv[r, sl]
            return 0

        lax.fori_loop(0, CHUNK * CSL, add_loop, 0, unroll=8)

        pltpu.async_copy(xv, oslice(j), sem_st)

        if b == B - 1 and t + 2 < N_CHUNKS:
            start_pload(t + 2)

        if j + LEAD < NJOBS:
            if j >= NXB - LEAD:
                wait_store((j + LEAD) % NXB)
            start_xload(j + LEAD)

    for j in range(NJOBS - NXB, NJOBS):
        wait_store(j % NXB)


_sc_call = pl.kernel(
    _sc_body,
    out_type=jax.ShapeDtypeStruct((B, S, D), jnp.float32),
    mesh=plsc.VectorSubcoreMesh(core_axis_name="c", subcore_axis_name="s"),
    scratch_types=[
        pltpu.VMEM((NXB, CHUNK, D), jnp.float32),
        pltpu.VMEM((2, CHUNK, D), jnp.float32),
        pltpu.SemaphoreType.DMA,
        pltpu.SemaphoreType.DMA,
        pltpu.SemaphoreType.DMA,
    ],
    compiler_params=pltpu.CompilerParams(use_tc_tiling_on_sc=True),
)


def kernel(x, pos_table):
    return _sc_call(x, pos_table)
